# baseline (speedup 1.0000x reference)
import functools

import jax
import jax.numpy as jnp
from jax import lax
from jax.experimental import pallas as pl
from jax.experimental.pallas import tpu as pltpu
from jax.experimental.pallas import tpu_sc as plsc

_BATCH = 16384
_FACTOR = 32
_NC = 2
_NS = 16
_NW = _NC * _NS
_BPW = _BATCH // _NW
_LANES = 16
_GROUP = 16


def _extract_col(strip, c):
    cv = jnp.full((_LANES,), c, jnp.int32)
    lo = plsc.load_gather(strip, [lax.iota(jnp.int32, _LANES), cv])
    hi = plsc.load_gather(strip, [lax.iota(jnp.int32, _LANES) + 16, cv])
    return lo, hi


def _body(user_hbm, item_i_hbm, item_j_hbm, eut_hbm, eit_hbm,
          pred_i_hbm, pred_j_hbm,
          uidx, iidx, jidx, su0, su1, si0, si1, sj0, sj1, oi, oj,
          sem0, sem1):
    sem = (sem0, sem1)
    wid = lax.axis_index("s") * _NC + lax.axis_index("c")
    base = wid * _BPW
    pltpu.sync_copy(user_hbm.at[pl.ds(base, _BPW)], uidx.at[pl.ds(0, _BPW)])
    pltpu.sync_copy(item_i_hbm.at[pl.ds(base, _BPW)], iidx.at[pl.ds(0, _BPW)])
    pltpu.sync_copy(item_j_hbm.at[pl.ds(base, _BPW)], jidx.at[pl.ds(0, _BPW)])

    su = (su0, su1)
    si = (si0, si1)
    sj = (sj0, sj1)

    def issue(ru, ri, rj, k):
        ualn = pl.multiple_of((ru // 128) * 128, 128)
        ialn = pl.multiple_of((ri // 128) * 128, 128)
        jaln = pl.multiple_of((rj // 128) * 128, 128)
        pltpu.async_copy(eut_hbm.at[:, pl.ds(ualn, 128)], su[k], sem[k])
        pltpu.async_copy(eit_hbm.at[:, pl.ds(ialn, 128)], si[k], sem[k])
        pltpu.async_copy(eit_hbm.at[:, pl.ds(jaln, 128)], sj[k], sem[k])

    def drain3(k):
        pltpu.make_async_copy(eut_hbm.at[:, pl.ds(0, 128)], su0, sem[k]).wait()
        pltpu.make_async_copy(eut_hbm.at[:, pl.ds(0, 128)], si0, sem[k]).wait()
        pltpu.make_async_copy(eut_hbm.at[:, pl.ds(0, 128)], sj0, sem[k]).wait()

    uv = uidx[pl.ds(0, _LANES)]
    iv = iidx[pl.ds(0, _LANES)]
    jv = jidx[pl.ds(0, _LANES)]
    issue(uv[0], iv[0], jv[0], 0)

    def group(gix, carry):
        b0 = gix * _GROUP
        ucur = uidx[pl.ds(b0, _LANES)]
        icur = iidx[pl.ds(b0, _LANES)]
        jcur = jidx[pl.ds(b0, _LANES)]
        unxt = uidx[pl.ds(b0 + _LANES, _LANES)]
        inxt = iidx[pl.ds(b0 + _LANES, _LANES)]
        jnxt = jidx[pl.ds(b0 + _LANES, _LANES)]
        acc_i = jnp.zeros((_LANES,), jnp.float32)
        acc_j = jnp.zeros((_LANES,), jnp.float32)
        lane = lax.iota(jnp.int32, _LANES)
        for g in range(_GROUP):
            b = b0 + g
            k = g % 2
            if g < _GROUP - 1:
                nu, ni, nj = ucur[g + 1], icur[g + 1], jcur[g + 1]
            else:
                nu, ni, nj = unxt[0], inxt[0], jnxt[0]

            @pl.when(b + 1 < _BPW)
            def _():
                issue(nu, ni, nj, (g + 1) % 2)

            drain3(k)
            u_lo, u_hi = _extract_col(su[k], lax.rem(ucur[g], 128))
            i_lo, i_hi = _extract_col(si[k], lax.rem(icur[g], 128))
            j_lo, j_hi = _extract_col(sj[k], lax.rem(jcur[g], 128))
            di = jnp.sum(u_lo * i_lo + u_hi * i_hi)
            dj = jnp.sum(u_lo * j_lo + u_hi * j_hi)
            sel = lane == g
            acc_i = jnp.where(sel, di, acc_i)
            acc_j = jnp.where(sel, dj, acc_j)
        oi[pl.ds(b0, _LANES)] = acc_i
        oj[pl.ds(b0, _LANES)] = acc_j
        return carry

    lax.fori_loop(0, _BPW // _GROUP, group, 0)

    pltpu.sync_copy(oi, pred_i_hbm.at[pl.ds(base, _BPW)])
    pltpu.sync_copy(oj, pred_j_hbm.at[pl.ds(base, _BPW)])


@jax.jit
def _mf_bpr(user, item_i, item_j, embed_user_t, embed_item_t):
    mesh = plsc.VectorSubcoreMesh(core_axis_name="c", subcore_axis_name="s")
    strip = functools.partial(pltpu.VMEM, (_FACTOR, 128), jnp.float32)
    run = pl.kernel(
        _body,
        out_type=(
            jax.ShapeDtypeStruct((_BATCH,), jnp.float32),
            jax.ShapeDtypeStruct((_BATCH,), jnp.float32),
        ),
        mesh=mesh,
        scratch_types=[
            pltpu.VMEM((_BPW + _LANES,), jnp.int32),
            pltpu.VMEM((_BPW + _LANES,), jnp.int32),
            pltpu.VMEM((_BPW + _LANES,), jnp.int32),
            strip(), strip(), strip(), strip(), strip(), strip(),
            pltpu.VMEM((_BPW,), jnp.float32),
            pltpu.VMEM((_BPW,), jnp.float32),
            pltpu.SemaphoreType.DMA,
            pltpu.SemaphoreType.DMA,
        ],
        compiler_params=pltpu.CompilerParams(
            needs_layout_passes=False, use_tc_tiling_on_sc=True),
    )
    return run(user, item_i, item_j, embed_user_t, embed_item_t)


def kernel(user, item_i, item_j, embed_user, embed_item):
    user = user.astype(jnp.int32)
    item_i = item_i.astype(jnp.int32)
    item_j = item_j.astype(jnp.int32)
    pred_i, pred_j = _mf_bpr(user, item_i, item_j,
                             embed_user.T, embed_item.T)
    return (pred_i, pred_j)
